# baseline (speedup 1.0000x reference)
import functools

import jax
import jax.numpy as jnp
from jax import lax
from jax.experimental import pallas as pl
from jax.experimental.pallas import tpu as pltpu
from jax.experimental.pallas import tpu_sc as plsc

_B = 16384
_D = 64
_NEG = 5
_EMB = 1000000
_ROOT = 100000
_NC = 2
_NS = 16
_NW = _NC * _NS
_L = 16
_BPW = _B // _NW
_C = 64
_NCHUNK = _BPW // _C
_GROUPS = _C // _L
_OC = 8


def _sc_scores_body(U1, V1, R1, alpha, pos_u, pos_v, negv_t, pos_r, out,
                    idx_u, idx_v, idx_r, idx_n0, idx_n1, idx_n2, idx_n3,
                    idx_n4, off_u, off_v, off_r, off_n0, off_n1, off_n2,
                    off_n3, off_n4, val_u, val_v, val_r, val_n0, val_n1,
                    val_n2, val_n3, val_n4, alpha_buf, out_buf, sem):
    idx_n = [idx_n0, idx_n1, idx_n2, idx_n3, idx_n4]
    off_n = [off_n0, off_n1, off_n2, off_n3, off_n4]
    val_n = [val_n0, val_n1, val_n2, val_n3, val_n4]

    wid = lax.axis_index("s") * _NC + lax.axis_index("c")
    ids0 = lax.iota(jnp.int32, _L)

    for c in range(_NCHUNK):
        base = pl.multiple_of(wid * _BPW + c * _C, _C)

        pltpu.sync_copy(pos_u.at[pl.ds(base, _C)], idx_u)
        pltpu.sync_copy(pos_v.at[pl.ds(base, _C)], idx_v)
        pltpu.sync_copy(pos_r.at[pl.ds(base, _C)], idx_r)
        for k in range(_NEG):
            pltpu.sync_copy(negv_t.at[pl.ds(k * _B + base, _C)], idx_n[k])

        def off_body(d, _):
            du = d * _EMB
            dr = d * _ROOT
            for j in range(_GROUPS):
                sl = pl.ds(j * _L, _L)
                dsl = pl.ds(d * _C + j * _L, _L)
                off_u[dsl] = idx_u[sl] + du
                off_v[dsl] = idx_v[sl] + du
                off_r[dsl] = idx_r[sl] + dr
                for k in range(_NEG):
                    off_n[k][dsl] = idx_n[k][sl] + du
            return 0

        lax.fori_loop(0, _D, off_body, 0)

        cps = [
            pltpu.async_copy(U1.at[off_u], val_u, sem),
            pltpu.async_copy(R1.at[off_r], val_r, sem),
            pltpu.async_copy(V1.at[off_v], val_v, sem),
            pltpu.async_copy(alpha.at[idx_u], alpha_buf, sem),
        ]
        for k in range(_NEG):
            cps.append(pltpu.async_copy(V1.at[off_n[k]], val_n[k], sem))
        for cp in cps:
            cp.wait()

        for g in range(_GROUPS):
            ids = ids0 + (g * _L)
            gsl = pl.ds(g * _L, _L)
            a = alpha_buf[gsl]
            a = jnp.minimum(jnp.maximum(a, 0.01), 0.99)

            def d_body(d, accs):
                dsl = pl.ds(d * _C + g * _L, _L)
                u = val_u[dsl]
                r = val_r[dsl]
                v = val_v[dsl]
                nxt = [accs[0] + u * v, accs[1] + r * v]
                for k in range(_NEG):
                    n = val_n[k][dsl]
                    nxt.append(accs[2 + 2 * k] + u * n)
                    nxt.append(accs[3 + 2 * k] + r * n)
                return tuple(nxt)

            zero = jnp.zeros((_L,), jnp.float32)
            accs = lax.fori_loop(0, _D, d_body, (zero,) * (2 * (1 + _NEG)))

            oidx = ids * _OC
            s = a * accs[0] + (1.0 - a) * accs[1]
            s = jnp.minimum(jnp.maximum(s, -10.0), 10.0)
            plsc.store_scatter(out_buf, [oidx], s)
            for k in range(_NEG):
                s = a * accs[2 + 2 * k] + (1.0 - a) * accs[3 + 2 * k]
                s = jnp.minimum(jnp.maximum(s, -10.0), 10.0)
                plsc.store_scatter(out_buf, [oidx + (k + 1)], s)

        pltpu.sync_copy(out_buf, out.at[pl.ds(base * _OC, _C * _OC)])


_sc_scores = functools.partial(
    pl.kernel,
    out_type=jax.ShapeDtypeStruct((_B * _OC,), jnp.float32),
    mesh=plsc.VectorSubcoreMesh(core_axis_name="c", subcore_axis_name="s"),
    scratch_types=(
        [pltpu.VMEM((_C,), jnp.int32) for _ in range(8)]
        + [pltpu.VMEM((_D * _C,), jnp.int32) for _ in range(8)]
        + [pltpu.VMEM((_D * _C,), jnp.float32) for _ in range(8)]
        + [pltpu.VMEM((_C,), jnp.float32),
           pltpu.VMEM((_C * _OC,), jnp.float32),
           pltpu.SemaphoreType.DMA]
    ),
    compiler_params=pltpu.CompilerParams(needs_layout_passes=False),
)(_sc_scores_body)


def _loss_body(s_ref, o_ref):
    s = s_ref[...]
    cid = lax.broadcasted_iota(jnp.int32, s.shape, 1) % _OC
    x = jnp.where(cid == 0, -s, s)
    sp = jnp.maximum(x, 0.0) + jnp.log1p(jnp.exp(-jnp.abs(x)))
    sp = jnp.where(cid < 1 + _NEG, sp, 0.0)
    o_ref[...] = (jnp.sum(sp) * (1.0 / _B)).reshape(1, 1)


def kernel(U, V, R, alpha, pos_u, pos_v, neg_v, pos_r):
    pos_u = pos_u.astype(jnp.int32)
    pos_v = pos_v.astype(jnp.int32)
    pos_r = pos_r.astype(jnp.int32)
    negv_t = jnp.transpose(neg_v.astype(jnp.int32)).reshape(_NEG * _B)

    U1 = jnp.transpose(U).reshape(_EMB * _D)
    V1 = jnp.transpose(V).reshape(_EMB * _D)
    R1 = jnp.transpose(R).reshape(_ROOT * _D)

    scores = _sc_scores(U1, V1, R1, alpha, pos_u, pos_v, negv_t, pos_r)
    scores2d = scores.reshape(_B * _OC // 128, 128)

    loss = pl.pallas_call(
        _loss_body,
        out_shape=jax.ShapeDtypeStruct((1, 1), jnp.float32),
    )(scores2d)
    return loss[0, 0]


# speedup vs baseline: 6.1444x; 6.1444x over previous
import functools

import jax
import jax.numpy as jnp
from jax import lax
from jax.experimental import pallas as pl
from jax.experimental.pallas import tpu as pltpu
from jax.experimental.pallas import tpu_sc as plsc

_B = 16384
_D = 64
_NEG = 5
_NC = 2
_NS = 16
_NW = _NC * _NS
_L = 16
_BPW = _B // _NW
_C = 128
_NCHUNK = _BPW // _C
_GROUPS = _C // _L
_OC = 8


def _sc_scores_body(U, V, R, alpha, pos_u, pos_v, negv_t, pos_r, out,
                    idx_u, idx_v, idx_r, idx_n0, idx_n1, idx_n2, idx_n3,
                    idx_n4, rows_u, rows_v, rows_r, rows_n0, rows_n1,
                    rows_n2, rows_n3, rows_n4, alpha_buf, out_buf, sem):
    idx_n = [idx_n0, idx_n1, idx_n2, idx_n3, idx_n4]
    rows_n = [rows_n0, rows_n1, rows_n2, rows_n3, rows_n4]

    wid = lax.axis_index("s") * _NC + lax.axis_index("c")
    ids0 = lax.iota(jnp.int32, _L)

    for c in range(_NCHUNK):
        base = pl.multiple_of(wid * _BPW + c * _C, _C)

        pltpu.sync_copy(pos_u.at[pl.ds(base, _C)], idx_u)
        pltpu.sync_copy(pos_v.at[pl.ds(base, _C)], idx_v)
        pltpu.sync_copy(pos_r.at[pl.ds(base, _C)], idx_r)
        for k in range(_NEG):
            pltpu.sync_copy(negv_t.at[pl.ds(k * _B + base, _C)], idx_n[k])

        cps = [
            pltpu.async_copy(U.at[idx_u], rows_u, sem),
            pltpu.async_copy(R.at[idx_r], rows_r, sem),
            pltpu.async_copy(V.at[idx_v], rows_v, sem),
            pltpu.async_copy(alpha.at[idx_u], alpha_buf, sem),
        ]
        for k in range(_NEG):
            cps.append(pltpu.async_copy(V.at[idx_n[k]], rows_n[k], sem))
        for cp in cps:
            cp.wait()

        for g in range(_GROUPS):
            ids = ids0 + (g * _L)
            a = alpha_buf[pl.ds(g * _L, _L)]
            a = jnp.minimum(jnp.maximum(a, 0.01), 0.99)

            def d_body(d, accs):
                dd = jnp.zeros((_L,), jnp.int32) + d
                u = plsc.load_gather(rows_u, [ids, dd])
                r = plsc.load_gather(rows_r, [ids, dd])
                v = plsc.load_gather(rows_v, [ids, dd])
                nxt = [accs[0] + u * v, accs[1] + r * v]
                for k in range(_NEG):
                    n = plsc.load_gather(rows_n[k], [ids, dd])
                    nxt.append(accs[2 + 2 * k] + u * n)
                    nxt.append(accs[3 + 2 * k] + r * n)
                return tuple(nxt)

            zero = jnp.zeros((_L,), jnp.float32)
            accs = lax.fori_loop(0, _D, d_body, (zero,) * (2 * (1 + _NEG)))

            oidx = ids * _OC
            s = a * accs[0] + (1.0 - a) * accs[1]
            s = jnp.minimum(jnp.maximum(s, -10.0), 10.0)
            plsc.store_scatter(out_buf, [oidx], s)
            for k in range(_NEG):
                s = a * accs[2 + 2 * k] + (1.0 - a) * accs[3 + 2 * k]
                s = jnp.minimum(jnp.maximum(s, -10.0), 10.0)
                plsc.store_scatter(out_buf, [oidx + (k + 1)], s)

        pltpu.sync_copy(out_buf, out.at[pl.ds(base * _OC, _C * _OC)])


_sc_scores = functools.partial(
    pl.kernel,
    out_type=jax.ShapeDtypeStruct((_B * _OC,), jnp.float32),
    mesh=plsc.VectorSubcoreMesh(core_axis_name="c", subcore_axis_name="s"),
    scratch_types=(
        [pltpu.VMEM((_C,), jnp.int32) for _ in range(8)]
        + [pltpu.VMEM((_C, _D), jnp.float32) for _ in range(8)]
        + [pltpu.VMEM((_C,), jnp.float32),
           pltpu.VMEM((_C * _OC,), jnp.float32),
           pltpu.SemaphoreType.DMA]
    ),
    compiler_params=pltpu.CompilerParams(needs_layout_passes=False,
                                         use_tc_tiling_on_sc=False),
)(_sc_scores_body)


def _tpose_body(src_ref, dst_ref):
    dst_ref[...] = src_ref[...].T


def _tc_transpose(xt, n):
    blk = 4096
    grid = (n + blk - 1) // blk
    return pl.pallas_call(
        _tpose_body,
        grid=(grid,),
        in_specs=[pl.BlockSpec((_D, blk), lambda b: (0, b))],
        out_specs=pl.BlockSpec((blk, _D), lambda b: (b, 0)),
        out_shape=jax.ShapeDtypeStruct((n, _D), jnp.float32),
    )(xt)


def _loss_body(s_ref, o_ref):
    s = s_ref[...]
    cid = lax.broadcasted_iota(jnp.int32, s.shape, 1) % _OC
    x = jnp.where(cid == 0, -s, s)
    sp = jnp.maximum(x, 0.0) + jnp.log1p(jnp.exp(-jnp.abs(x)))
    sp = jnp.where(cid < 1 + _NEG, sp, 0.0)
    o_ref[...] = (jnp.sum(sp) * (1.0 / _B)).reshape(1, 1)


def kernel(U, V, R, alpha, pos_u, pos_v, neg_v, pos_r):
    pos_u = pos_u.astype(jnp.int32)
    pos_v = pos_v.astype(jnp.int32)
    pos_r = pos_r.astype(jnp.int32)
    negv_t = jnp.transpose(neg_v.astype(jnp.int32)).reshape(_NEG * _B)

    Urm = _tc_transpose(jnp.transpose(U), U.shape[0])
    Vrm = _tc_transpose(jnp.transpose(V), V.shape[0])
    Rrm = _tc_transpose(jnp.transpose(R), R.shape[0])

    scores = _sc_scores(Urm, Vrm, Rrm, alpha, pos_u, pos_v, negv_t, pos_r)
    scores2d = scores.reshape(_B * _OC // 128, 128)

    loss = pl.pallas_call(
        _loss_body,
        out_shape=jax.ShapeDtypeStruct((1, 1), jnp.float32),
    )(scores2d)
    return loss[0, 0]


# speedup vs baseline: 11.6011x; 1.8881x over previous
import functools

import jax
import jax.numpy as jnp
from jax import lax
from jax.experimental import pallas as pl
from jax.experimental.pallas import tpu as pltpu
from jax.experimental.pallas import tpu_sc as plsc

_B = 16384
_D = 64
_NEG = 5
_NC = 2
_NS = 16
_NW = _NC * _NS
_L = 16
_BPW = _B // _NW
_C = 64
_NCHUNK = _BPW // _C
_GROUPS = _C // _L
_OC = 8
_AP = 1000064


def _sc_scores_body(U2, V2, R2, A2, pos_u, pos_v, negv_t, pos_r, out,
                    idx_u, idx_v, idx_r, idx_n0, idx_n1, idx_n2, idx_n3,
                    idx_n4, row_a, rows_u, rows_v, rows_r, rows_a,
                    rows_n0, rows_n1, rows_n2, rows_n3, rows_n4,
                    out_buf, sem):
    idx_n = [idx_n0, idx_n1, idx_n2, idx_n3, idx_n4]
    rows_n = [rows_n0, rows_n1, rows_n2, rows_n3, rows_n4]

    wid = lax.axis_index("s") * _NC + lax.axis_index("c")
    ids0 = lax.iota(jnp.int32, _L)

    for c in range(_NCHUNK):
        base = pl.multiple_of(wid * _BPW + c * _C, _C)

        pltpu.sync_copy(pos_u.at[pl.ds(base, _C)], idx_u)
        pltpu.sync_copy(pos_v.at[pl.ds(base, _C)], idx_v)
        pltpu.sync_copy(pos_r.at[pl.ds(base, _C)], idx_r)
        for k in range(_NEG):
            pltpu.sync_copy(negv_t.at[pl.ds(k * _B + base, _C)], idx_n[k])

        for j in range(_GROUPS):
            sl = pl.ds(j * _L, _L)
            row_a[sl] = lax.shift_right_logical(idx_u[sl], 7)

        cps = [
            pltpu.async_copy(U2.at[idx_u], rows_u, sem),
            pltpu.async_copy(R2.at[idx_r], rows_r, sem),
            pltpu.async_copy(V2.at[idx_v], rows_v, sem),
            pltpu.async_copy(A2.at[row_a], rows_a, sem),
        ]
        for k in range(_NEG):
            cps.append(pltpu.async_copy(V2.at[idx_n[k]], rows_n[k], sem))
        for cp in cps:
            cp.wait()

        for g in range(_GROUPS):
            ids = ids0 + (g * _L)
            sl = pl.ds(g * _L, _L)
            lane_a = jnp.bitwise_and(idx_u[sl], 127)
            a = plsc.load_gather(rows_a, [ids, lane_a])
            a = jnp.minimum(jnp.maximum(a, 0.01), 0.99)

            def d_body(d, accs):
                dd = jnp.zeros((_L,), jnp.int32) + d
                u = plsc.load_gather(rows_u, [ids, dd])
                r = plsc.load_gather(rows_r, [ids, dd])
                v = plsc.load_gather(rows_v, [ids, dd])
                nxt = [accs[0] + u * v, accs[1] + r * v]
                for k in range(_NEG):
                    n = plsc.load_gather(rows_n[k], [ids, dd])
                    nxt.append(accs[2 + 2 * k] + u * n)
                    nxt.append(accs[3 + 2 * k] + r * n)
                return tuple(nxt)

            zero = jnp.zeros((_L,), jnp.float32)
            accs = lax.fori_loop(0, _D, d_body, (zero,) * (2 * (1 + _NEG)))

            oidx = ids * _OC
            s = a * accs[0] + (1.0 - a) * accs[1]
            s = jnp.minimum(jnp.maximum(s, -10.0), 10.0)
            plsc.store_scatter(out_buf, [oidx], s)
            for k in range(_NEG):
                s = a * accs[2 + 2 * k] + (1.0 - a) * accs[3 + 2 * k]
                s = jnp.minimum(jnp.maximum(s, -10.0), 10.0)
                plsc.store_scatter(out_buf, [oidx + (k + 1)], s)

        pltpu.sync_copy(out_buf, out.at[pl.ds(base * _OC, _C * _OC)])


_sc_scores = functools.partial(
    pl.kernel,
    out_type=jax.ShapeDtypeStruct((_B * _OC,), jnp.float32),
    mesh=plsc.VectorSubcoreMesh(core_axis_name="c", subcore_axis_name="s"),
    scratch_types=(
        [pltpu.VMEM((_C,), jnp.int32) for _ in range(8)]
        + [pltpu.VMEM((_C,), jnp.int32)]
        + [pltpu.VMEM((_C, 128), jnp.float32) for _ in range(9)]
        + [pltpu.VMEM((_C * _OC,), jnp.float32),
           pltpu.SemaphoreType.DMA]
    ),
    compiler_params=pltpu.CompilerParams(needs_layout_passes=False,
                                         use_tc_tiling_on_sc=True),
)(_sc_scores_body)


_TBLK = 8192


def _tpose_body(src_ref, dst_ref):
    x = src_ref[...]
    y = x.T
    dst_ref[...] = jnp.concatenate([y, y], axis=1)


def _tc_transpose(xt, n):
    grid = (n + _TBLK - 1) // _TBLK
    return pl.pallas_call(
        _tpose_body,
        grid=(grid,),
        in_specs=[pl.BlockSpec((_D, _TBLK), lambda b: (0, b))],
        out_specs=pl.BlockSpec((_TBLK, 128), lambda b: (b, 0)),
        out_shape=jax.ShapeDtypeStruct((n, 128), jnp.float32),
    )(xt)


def _loss_body(s_ref, o_ref):
    s = s_ref[...]
    cid = lax.broadcasted_iota(jnp.int32, s.shape, 1) % _OC
    x = jnp.where(cid == 0, -s, s)
    sp = jnp.maximum(x, 0.0) + jnp.log1p(jnp.exp(-jnp.abs(x)))
    sp = jnp.where(cid < 1 + _NEG, sp, 0.0)
    o_ref[...] = (jnp.sum(sp) * (1.0 / _B)).reshape(1, 1)


def kernel(U, V, R, alpha, pos_u, pos_v, neg_v, pos_r):
    pos_u = pos_u.astype(jnp.int32)
    pos_v = pos_v.astype(jnp.int32)
    pos_r = pos_r.astype(jnp.int32)
    negv_t = jnp.transpose(neg_v.astype(jnp.int32)).reshape(_NEG * _B)

    U2 = _tc_transpose(jnp.transpose(U), U.shape[0])
    V2 = _tc_transpose(jnp.transpose(V), V.shape[0])
    R2 = _tc_transpose(jnp.transpose(R), R.shape[0])
    A2 = jnp.pad(alpha, (0, _AP - alpha.shape[0])).reshape(_AP // 128, 128)

    scores = _sc_scores(U2, V2, R2, A2, pos_u, pos_v, negv_t, pos_r)
    scores2d = scores.reshape(_B * _OC // 128, 128)

    loss = pl.pallas_call(
        _loss_body,
        out_shape=jax.ShapeDtypeStruct((1, 1), jnp.float32),
    )(scores2d)
    return loss[0, 0]


# speedup vs baseline: 12.3488x; 1.0645x over previous
import functools

import jax
import jax.numpy as jnp
from jax import lax
from jax.experimental import pallas as pl
from jax.experimental.pallas import tpu as pltpu
from jax.experimental.pallas import tpu_sc as plsc

_B = 16384
_D = 64
_NEG = 5
_NC = 2
_NS = 16
_NW = _NC * _NS
_L = 16
_BPW = _B // _NW
_C = 32
_NCHUNK = _BPW // _C
_GROUPS = _C // _L
_OC = 8


def _sc_scores_body(U2, V2, R2, alpha, pos_u, pos_v, negv_t, pos_r, out,
                    idx_u, idx_v, idx_r, idx_n0, idx_n1, idx_n2, idx_n3,
                    idx_n4, ru0, rv0, rr0, rn00, rn10, rn20, rn30, rn40,
                    ru1, rv1, rr1, rn01, rn11, rn21, rn31, rn41,
                    alpha_buf, out_buf, sem0, sem1, sema):
    idx_n = [idx_n0, idx_n1, idx_n2, idx_n3, idx_n4]
    rows = [[ru0, rr0, rv0, rn00, rn10, rn20, rn30, rn40],
            [ru1, rr1, rv1, rn01, rn11, rn21, rn31, rn41]]
    sems = [sem0, sem1]

    wid = lax.axis_index("s") * _NC + lax.axis_index("c")
    ids0 = lax.iota(jnp.int32, _L)
    wbase = pl.multiple_of(wid * _BPW, _BPW)

    pltpu.sync_copy(pos_u.at[pl.ds(wbase, _BPW)], idx_u)
    pltpu.sync_copy(pos_v.at[pl.ds(wbase, _BPW)], idx_v)
    pltpu.sync_copy(pos_r.at[pl.ds(wbase, _BPW)], idx_r)
    for k in range(_NEG):
        pltpu.sync_copy(negv_t.at[pl.ds(k * _B + wbase, _BPW)], idx_n[k])
    alpha_cp = pltpu.async_copy(alpha.at[idx_u], alpha_buf, sema)

    def fire(c):
        p = c % 2
        csl = pl.ds(c * _C, _C)
        cps = [
            pltpu.async_copy(U2.at[idx_u.at[csl]], rows[p][0], sems[p]),
            pltpu.async_copy(R2.at[idx_r.at[csl]], rows[p][1], sems[p]),
            pltpu.async_copy(V2.at[idx_v.at[csl]], rows[p][2], sems[p]),
        ]
        for k in range(_NEG):
            cps.append(pltpu.async_copy(V2.at[idx_n[k].at[csl]],
                                        rows[p][3 + k], sems[p]))
        return cps

    pend = fire(0)
    alpha_cp.wait()

    for c in range(_NCHUNK):
        nxt_pend = fire(c + 1) if c + 1 < _NCHUNK else []
        for cp in pend:
            cp.wait()
        pend = nxt_pend
        p = c % 2
        ru, rr, rv = rows[p][0], rows[p][1], rows[p][2]
        rn = rows[p][3:]

        for g in range(_GROUPS):
            ids = ids0 + (g * _L)
            a = alpha_buf[pl.ds(c * _C + g * _L, _L)]
            a = jnp.minimum(jnp.maximum(a, 0.01), 0.99)

            def d_body(d, accs):
                dd = jnp.zeros((_L,), jnp.int32) + d
                u = plsc.load_gather(ru, [ids, dd])
                r = plsc.load_gather(rr, [ids, dd])
                v = plsc.load_gather(rv, [ids, dd])
                nxt = [accs[0] + u * v, accs[1] + r * v]
                for k in range(_NEG):
                    n = plsc.load_gather(rn[k], [ids, dd])
                    nxt.append(accs[2 + 2 * k] + u * n)
                    nxt.append(accs[3 + 2 * k] + r * n)
                return tuple(nxt)

            zero = jnp.zeros((_L,), jnp.float32)
            accs = lax.fori_loop(0, _D, d_body, (zero,) * (2 * (1 + _NEG)))

            oidx = ids * _OC + ((c * _C + g * _L) * _OC)
            s = a * accs[0] + (1.0 - a) * accs[1]
            s = jnp.minimum(jnp.maximum(s, -10.0), 10.0)
            plsc.store_scatter(out_buf, [oidx], s)
            for k in range(_NEG):
                s = a * accs[2 + 2 * k] + (1.0 - a) * accs[3 + 2 * k]
                s = jnp.minimum(jnp.maximum(s, -10.0), 10.0)
                plsc.store_scatter(out_buf, [oidx + (k + 1)], s)

    pltpu.sync_copy(out_buf, out.at[pl.ds(wbase * _OC, _BPW * _OC)])


_sc_scores = functools.partial(
    pl.kernel,
    out_type=jax.ShapeDtypeStruct((_B * _OC,), jnp.float32),
    mesh=plsc.VectorSubcoreMesh(core_axis_name="c", subcore_axis_name="s"),
    scratch_types=(
        [pltpu.VMEM((_BPW,), jnp.int32) for _ in range(8)]
        + [pltpu.VMEM((_C, 128), jnp.float32) for _ in range(16)]
        + [pltpu.VMEM((_BPW,), jnp.float32),
           pltpu.VMEM((_BPW * _OC,), jnp.float32),
           pltpu.SemaphoreType.DMA, pltpu.SemaphoreType.DMA,
           pltpu.SemaphoreType.DMA]
    ),
    compiler_params=pltpu.CompilerParams(needs_layout_passes=False,
                                         use_tc_tiling_on_sc=True),
)(_sc_scores_body)


_TBLK = 8192


def _tpose_body(src_ref, dst_ref):
    x = src_ref[...]
    y = x.T
    dst_ref[...] = jnp.concatenate([y, y], axis=1)


def _tc_transpose(xt, n):
    grid = (n + _TBLK - 1) // _TBLK
    return pl.pallas_call(
        _tpose_body,
        grid=(grid,),
        in_specs=[pl.BlockSpec((_D, _TBLK), lambda b: (0, b))],
        out_specs=pl.BlockSpec((_TBLK, 128), lambda b: (b, 0)),
        out_shape=jax.ShapeDtypeStruct((n, 128), jnp.float32),
    )(xt)


def _loss_body(s_ref, o_ref):
    s = s_ref[...]
    cid = lax.broadcasted_iota(jnp.int32, s.shape, 1) % _OC
    x = jnp.where(cid == 0, -s, s)
    sp = jnp.maximum(x, 0.0) + jnp.log1p(jnp.exp(-jnp.abs(x)))
    sp = jnp.where(cid < 1 + _NEG, sp, 0.0)
    o_ref[...] = (jnp.sum(sp) * (1.0 / _B)).reshape(1, 1)


def kernel(U, V, R, alpha, pos_u, pos_v, neg_v, pos_r):
    pos_u = pos_u.astype(jnp.int32)
    pos_v = pos_v.astype(jnp.int32)
    pos_r = pos_r.astype(jnp.int32)
    negv_t = jnp.transpose(neg_v.astype(jnp.int32)).reshape(_NEG * _B)

    U2 = _tc_transpose(jnp.transpose(U), U.shape[0])
    V2 = _tc_transpose(jnp.transpose(V), V.shape[0])
    R2 = _tc_transpose(jnp.transpose(R), R.shape[0])

    scores = _sc_scores(U2, V2, R2, alpha, pos_u, pos_v, negv_t, pos_r)
    scores2d = scores.reshape(_B * _OC // 128, 128)

    loss = pl.pallas_call(
        _loss_body,
        out_shape=jax.ShapeDtypeStruct((1, 1), jnp.float32),
    )(scores2d)
    return loss[0, 0]


# speedup vs baseline: 12.3571x; 1.0007x over previous
import functools

import jax
import jax.numpy as jnp
from jax import lax
from jax.experimental import pallas as pl
from jax.experimental.pallas import tpu as pltpu
from jax.experimental.pallas import tpu_sc as plsc

_B = 16384
_D = 64
_NEG = 5
_NC = 2
_NS = 16
_NW = _NC * _NS
_L = 16
_BPW = _B // _NW
_C = 32
_NCHUNK = _BPW // _C
_GROUPS = _C // _L
_OC = 8


def _sc_scores_body(U2, V2, R2, alpha, pos_u, pos_v, negv_t, pos_r, out,
                    idx_u, idx_v, idx_r, idx_n0, idx_n1, idx_n2, idx_n3,
                    idx_n4, ru0, rv0, rr0, rn00, rn10, rn20, rn30, rn40,
                    ru1, rv1, rr1, rn01, rn11, rn21, rn31, rn41,
                    ic00, ic01, ic02, ic03, ic04, ic05, ic06, ic07,
                    ic10, ic11, ic12, ic13, ic14, ic15, ic16, ic17,
                    alpha_buf, out_buf, sem0, sem1, sema):
    idx_n = [idx_n0, idx_n1, idx_n2, idx_n3, idx_n4]
    idx_all = [idx_u, idx_r, idx_v] + idx_n
    rows = [[ru0, rr0, rv0, rn00, rn10, rn20, rn30, rn40],
            [ru1, rr1, rv1, rn01, rn11, rn21, rn31, rn41]]
    idxc = [[ic00, ic01, ic02, ic03, ic04, ic05, ic06, ic07],
            [ic10, ic11, ic12, ic13, ic14, ic15, ic16, ic17]]
    sems = [sem0, sem1]

    wid = lax.axis_index("s") * _NC + lax.axis_index("c")
    ids0 = lax.iota(jnp.int32, _L)
    wbase = pl.multiple_of(wid * _BPW, _BPW)

    pltpu.sync_copy(pos_u.at[pl.ds(wbase, _BPW)], idx_u)
    pltpu.sync_copy(pos_v.at[pl.ds(wbase, _BPW)], idx_v)
    pltpu.sync_copy(pos_r.at[pl.ds(wbase, _BPW)], idx_r)
    for k in range(_NEG):
        pltpu.sync_copy(negv_t.at[pl.ds(k * _B + wbase, _BPW)], idx_n[k])
    alpha_cp = pltpu.async_copy(alpha.at[idx_u], alpha_buf, sema)

    def fire(c):
        p = c % 2
        for t in range(8):
            for q in range(_GROUPS):
                qsl = pl.ds(q * _L, _L)
                idxc[p][t][qsl] = idx_all[t][pl.ds(c * _C + q * _L, _L)]
        tables = [U2, R2, V2] + [V2] * _NEG
        return [pltpu.async_copy(tables[t].at[idxc[p][t]], rows[p][t],
                                 sems[p]) for t in range(8)]

    pend = fire(0)
    alpha_cp.wait()

    for c in range(_NCHUNK):
        nxt_pend = fire(c + 1) if c + 1 < _NCHUNK else []
        for cp in pend:
            cp.wait()
        pend = nxt_pend
        p = c % 2
        ru, rr, rv = rows[p][0], rows[p][1], rows[p][2]
        rn = rows[p][3:]

        for g in range(_GROUPS):
            ids = ids0 + (g * _L)
            a = alpha_buf[pl.ds(c * _C + g * _L, _L)]
            a = jnp.minimum(jnp.maximum(a, 0.01), 0.99)

            def d_body(d, accs):
                dd = jnp.zeros((_L,), jnp.int32) + d
                u = plsc.load_gather(ru, [ids, dd])
                r = plsc.load_gather(rr, [ids, dd])
                v = plsc.load_gather(rv, [ids, dd])
                nxt = [accs[0] + u * v, accs[1] + r * v]
                for k in range(_NEG):
                    n = plsc.load_gather(rn[k], [ids, dd])
                    nxt.append(accs[2 + 2 * k] + u * n)
                    nxt.append(accs[3 + 2 * k] + r * n)
                return tuple(nxt)

            zero = jnp.zeros((_L,), jnp.float32)
            accs = lax.fori_loop(0, _D, d_body, (zero,) * (2 * (1 + _NEG)))

            oidx = ids * _OC + ((c * _C + g * _L) * _OC)
            s = a * accs[0] + (1.0 - a) * accs[1]
            s = jnp.minimum(jnp.maximum(s, -10.0), 10.0)
            plsc.store_scatter(out_buf, [oidx], s)
            for k in range(_NEG):
                s = a * accs[2 + 2 * k] + (1.0 - a) * accs[3 + 2 * k]
                s = jnp.minimum(jnp.maximum(s, -10.0), 10.0)
                plsc.store_scatter(out_buf, [oidx + (k + 1)], s)

    pltpu.sync_copy(out_buf, out.at[pl.ds(wbase * _OC, _BPW * _OC)])


_sc_scores = functools.partial(
    pl.kernel,
    out_type=jax.ShapeDtypeStruct((_B * _OC,), jnp.float32),
    mesh=plsc.VectorSubcoreMesh(core_axis_name="c", subcore_axis_name="s"),
    scratch_types=(
        [pltpu.VMEM((_BPW,), jnp.int32) for _ in range(8)]
        + [pltpu.VMEM((_C, 128), jnp.float32) for _ in range(16)]
        + [pltpu.VMEM((_C,), jnp.int32) for _ in range(16)]
        + [pltpu.VMEM((_BPW,), jnp.float32),
           pltpu.VMEM((_BPW * _OC,), jnp.float32),
           pltpu.SemaphoreType.DMA, pltpu.SemaphoreType.DMA,
           pltpu.SemaphoreType.DMA]
    ),
    compiler_params=pltpu.CompilerParams(needs_layout_passes=False,
                                         use_tc_tiling_on_sc=True),
)(_sc_scores_body)


_TBLK = 8192


def _tpose_body(src_ref, dst_ref):
    x = src_ref[...]
    y = x.T
    dst_ref[...] = jnp.concatenate([y, y], axis=1)


def _tc_transpose(xt, n):
    grid = (n + _TBLK - 1) // _TBLK
    return pl.pallas_call(
        _tpose_body,
        grid=(grid,),
        in_specs=[pl.BlockSpec((_D, _TBLK), lambda b: (0, b))],
        out_specs=pl.BlockSpec((_TBLK, 128), lambda b: (b, 0)),
        out_shape=jax.ShapeDtypeStruct((n, 128), jnp.float32),
    )(xt)


def _loss_body(s_ref, o_ref):
    s = s_ref[...]
    cid = lax.broadcasted_iota(jnp.int32, s.shape, 1) % _OC
    x = jnp.where(cid == 0, -s, s)
    sp = jnp.maximum(x, 0.0) + jnp.log1p(jnp.exp(-jnp.abs(x)))
    sp = jnp.where(cid < 1 + _NEG, sp, 0.0)
    o_ref[...] = (jnp.sum(sp) * (1.0 / _B)).reshape(1, 1)


def kernel(U, V, R, alpha, pos_u, pos_v, neg_v, pos_r):
    pos_u = pos_u.astype(jnp.int32)
    pos_v = pos_v.astype(jnp.int32)
    pos_r = pos_r.astype(jnp.int32)
    negv_t = jnp.transpose(neg_v.astype(jnp.int32)).reshape(_NEG * _B)

    U2 = _tc_transpose(jnp.transpose(U), U.shape[0])
    V2 = _tc_transpose(jnp.transpose(V), V.shape[0])
    R2 = _tc_transpose(jnp.transpose(R), R.shape[0])

    scores = _sc_scores(U2, V2, R2, alpha, pos_u, pos_v, negv_t, pos_r)
    scores2d = scores.reshape(_B * _OC // 128, 128)

    loss = pl.pallas_call(
        _loss_body,
        out_shape=jax.ShapeDtypeStruct((1, 1), jnp.float32),
    )(scores2d)
    return loss[0, 0]


# speedup vs baseline: 14.6382x; 1.1846x over previous
import functools

import jax
import jax.numpy as jnp
from jax import lax
from jax.experimental import pallas as pl
from jax.experimental.pallas import tpu as pltpu
from jax.experimental.pallas import tpu_sc as plsc

_B = 16384
_D = 64
_NEG = 5
_NC = 2
_NS = 16
_NW = _NC * _NS
_L = 16
_BPW = _B // _NW
_C = 32
_NCHUNK = _BPW // _C
_GROUPS = _C // _L
_OC = 8
_HN = 500000
_HNR = 50000


def _sc_scores_body(U2, V2, R2, alpha, pos_u, pos_v, negv_t, pos_r, out,
                    idx_u, idx_v, idx_r, idx_n0, idx_n1, idx_n2, idx_n3,
                    idx_n4, ru0, rv0, rr0, rn00, rn10, rn20, rn30, rn40,
                    ru1, rv1, rr1, rn01, rn11, rn21, rn31, rn41,
                    ic00, ic01, ic02, ic03, ic04, ic05, ic06, ic07,
                    ic10, ic11, ic12, ic13, ic14, ic15, ic16, ic17,
                    alpha_buf, out_buf, sem0, sem1, sema):
    idx_n = [idx_n0, idx_n1, idx_n2, idx_n3, idx_n4]
    idx_all = [idx_u, idx_r, idx_v] + idx_n
    rows = [[ru0, rr0, rv0, rn00, rn10, rn20, rn30, rn40],
            [ru1, rr1, rv1, rn01, rn11, rn21, rn31, rn41]]
    idxc = [[ic00, ic01, ic02, ic03, ic04, ic05, ic06, ic07],
            [ic10, ic11, ic12, ic13, ic14, ic15, ic16, ic17]]
    sems = [sem0, sem1]

    wid = lax.axis_index("s") * _NC + lax.axis_index("c")
    ids0 = lax.iota(jnp.int32, _L)
    wbase = pl.multiple_of(wid * _BPW, _BPW)

    pltpu.sync_copy(pos_u.at[pl.ds(wbase, _BPW)], idx_u)
    pltpu.sync_copy(pos_v.at[pl.ds(wbase, _BPW)], idx_v)
    pltpu.sync_copy(pos_r.at[pl.ds(wbase, _BPW)], idx_r)
    for k in range(_NEG):
        pltpu.sync_copy(negv_t.at[pl.ds(k * _B + wbase, _BPW)], idx_n[k])
    alpha_cp = pltpu.async_copy(alpha.at[idx_u], alpha_buf, sema)

    def fire(c):
        p = c % 2
        for t in range(8):
            for q in range(_GROUPS):
                qsl = pl.ds(q * _L, _L)
                iv = idx_all[t][pl.ds(c * _C + q * _L, _L)]
                idxc[p][t][qsl] = jnp.bitwise_or(
                    jnp.bitwise_and(iv, _W - 1),
                    lax.shift_left(lax.shift_right_logical(iv, 13), 12))
        tables = [U2, R2, V2] + [V2] * _NEG
        return [pltpu.async_copy(tables[t].at[idxc[p][t]], rows[p][t],
                                 sems[p]) for t in range(8)]

    pend = fire(0)
    alpha_cp.wait()

    for c in range(_NCHUNK):
        nxt_pend = fire(c + 1) if c + 1 < _NCHUNK else []
        for cp in pend:
            cp.wait()
        pend = nxt_pend
        p = c % 2
        ru, rr, rv = rows[p][0], rows[p][1], rows[p][2]
        rn = rows[p][3:]

        for g in range(_GROUPS):
            ids = ids0 + (g * _L)
            gsl = pl.ds(c * _C + g * _L, _L)
            a = alpha_buf[gsl]
            a = jnp.minimum(jnp.maximum(a, 0.01), 0.99)
            cols = [lax.shift_left(jnp.bitwise_and(
                        lax.shift_right_logical(idx_all[t][gsl], 12), 1), 6)
                    for t in range(8)]

            def d_body(d, accs):
                dd = jnp.zeros((_L,), jnp.int32) + d
                u = plsc.load_gather(ru, [ids, cols[0] + dd])
                r = plsc.load_gather(rr, [ids, cols[1] + dd])
                v = plsc.load_gather(rv, [ids, cols[2] + dd])
                nxt = [accs[0] + u * v, accs[1] + r * v]
                for k in range(_NEG):
                    n = plsc.load_gather(rn[k], [ids, cols[3 + k] + dd])
                    nxt.append(accs[2 + 2 * k] + u * n)
                    nxt.append(accs[3 + 2 * k] + r * n)
                return tuple(nxt)

            zero = jnp.zeros((_L,), jnp.float32)
            accs = lax.fori_loop(0, _D, d_body, (zero,) * (2 * (1 + _NEG)))

            oidx = ids * _OC + ((c * _C + g * _L) * _OC)
            s = a * accs[0] + (1.0 - a) * accs[1]
            s = jnp.minimum(jnp.maximum(s, -10.0), 10.0)
            plsc.store_scatter(out_buf, [oidx], s)
            for k in range(_NEG):
                s = a * accs[2 + 2 * k] + (1.0 - a) * accs[3 + 2 * k]
                s = jnp.minimum(jnp.maximum(s, -10.0), 10.0)
                plsc.store_scatter(out_buf, [oidx + (k + 1)], s)

    pltpu.sync_copy(out_buf, out.at[pl.ds(wbase * _OC, _BPW * _OC)])


_sc_scores = functools.partial(
    pl.kernel,
    out_type=jax.ShapeDtypeStruct((_B * _OC,), jnp.float32),
    mesh=plsc.VectorSubcoreMesh(core_axis_name="c", subcore_axis_name="s"),
    scratch_types=(
        [pltpu.VMEM((_BPW,), jnp.int32) for _ in range(8)]
        + [pltpu.VMEM((_C, 128), jnp.float32) for _ in range(16)]
        + [pltpu.VMEM((_C,), jnp.int32) for _ in range(16)]
        + [pltpu.VMEM((_BPW,), jnp.float32),
           pltpu.VMEM((_BPW * _OC,), jnp.float32),
           pltpu.SemaphoreType.DMA, pltpu.SemaphoreType.DMA,
           pltpu.SemaphoreType.DMA]
    ),
    compiler_params=pltpu.CompilerParams(needs_layout_passes=False,
                                         use_tc_tiling_on_sc=True),
)(_sc_scores_body)


_W = 4096


def _tpose_body(srca_ref, srcb_ref, dst_ref):
    dst_ref[...] = jnp.concatenate([srca_ref[...].T, srcb_ref[...].T],
                                   axis=1)


def _tc_transpose(xt, n):
    nq = (n + _W - 1) // _W
    nb = (nq + 1) // 2
    qmax = nq - 1
    return pl.pallas_call(
        _tpose_body,
        grid=(nb,),
        in_specs=[
            pl.BlockSpec((_D, _W), lambda b: (0, 2 * b)),
            pl.BlockSpec((_D, _W), lambda b: (0, jnp.minimum(2 * b + 1,
                                                             qmax))),
        ],
        out_specs=pl.BlockSpec((_W, 128), lambda b: (b, 0)),
        out_shape=jax.ShapeDtypeStruct((nb * _W, 128), jnp.float32),
    )(xt, xt)


def _loss_body(s_ref, o_ref):
    s = s_ref[...]
    cid = lax.broadcasted_iota(jnp.int32, s.shape, 1) % _OC
    x = jnp.where(cid == 0, -s, s)
    sp = jnp.maximum(x, 0.0) + jnp.log1p(jnp.exp(-jnp.abs(x)))
    sp = jnp.where(cid < 1 + _NEG, sp, 0.0)
    o_ref[...] = (jnp.sum(sp) * (1.0 / _B)).reshape(1, 1)


def kernel(U, V, R, alpha, pos_u, pos_v, neg_v, pos_r):
    pos_u = pos_u.astype(jnp.int32)
    pos_v = pos_v.astype(jnp.int32)
    pos_r = pos_r.astype(jnp.int32)
    negv_t = jnp.transpose(neg_v.astype(jnp.int32)).reshape(_NEG * _B)

    U2 = _tc_transpose(jnp.transpose(U), U.shape[0])
    V2 = _tc_transpose(jnp.transpose(V), V.shape[0])
    R2 = _tc_transpose(jnp.transpose(R), R.shape[0])

    scores = _sc_scores(U2, V2, R2, alpha, pos_u, pos_v, negv_t, pos_r)
    scores2d = scores.reshape(_B * _OC // 128, 128)

    loss = pl.pallas_call(
        _loss_body,
        out_shape=jax.ShapeDtypeStruct((1, 1), jnp.float32),
    )(scores2d)
    return loss[0, 0]
